# baseline (speedup 1.0000x reference)
import jax
import jax.numpy as jnp
import numpy as np
from jax import lax
from jax.experimental import pallas as pl
from jax.experimental.pallas import tpu as pltpu
from jax.experimental.pallas import tpu_sc as plsc

_GAMMA = 0.047
_NUM_BINS = 15
_BOUNDS = np.linspace(0.0, 1.0, _NUM_BINS + 1).astype(np.float32)
_LOWERS = _BOUNDS[:-1]
_UPPERS = _BOUNDS[1:]

_B = 16384
_C = 1000
_RB = 512


def _stats_kernel(x_ref, y_ref, conf_ref, acc_ref, tl_ref, btab_ref):
    i = pl.program_id(0)
    xv = x_ref[...]
    yv = y_ref[...]
    m = jnp.max(xv, axis=1, keepdims=True)
    e = jnp.exp(xv - m)
    s = jnp.sum(e, axis=1, keepdims=True)
    conf = 1.0 / s
    col = jax.lax.broadcasted_iota(jnp.int32, xv.shape, 1)
    xy = jnp.sum(jnp.where(col == yv, xv, 0.0), axis=1, keepdims=True)
    accv = (xy == m).astype(jnp.float32)
    tl = xy - m - jnp.log(s)
    conf_ref[...] = conf
    acc_ref[...] = accv
    tl_ref[...] = tl

    cnt = jnp.zeros_like(conf, dtype=jnp.int32)
    for lo in _LOWERS:
        cnt = cnt + (conf > float(lo)).astype(jnp.int32)
    idx = cnt - 1
    lane = jax.lax.broadcasted_iota(jnp.int32, (_RB, 128), 1)
    onehot = (lane == idx).astype(jnp.float32)
    n_p = jnp.sum(onehot, axis=0, keepdims=True)
    sa_p = jnp.sum(onehot * accv, axis=0, keepdims=True)
    sc_p = jnp.sum(onehot * conf, axis=0, keepdims=True)
    groups = [
        jnp.broadcast_to(jnp.transpose(v[:, :16], (1, 0)), (16, 16))
        for v in (n_p, sa_p, sc_p)
    ]
    upd = jnp.concatenate(groups, axis=0)

    @pl.when(i == 0)
    def _init():
        btab_ref[...] = upd

    @pl.when(i != 0)
    def _accum():
        btab_ref[...] += upd


def _row_stats(x, y):
    grid = _B // _RB
    return pl.pallas_call(
        _stats_kernel,
        grid=(grid,),
        in_specs=[
            pl.BlockSpec((_RB, _C), lambda i: (i, 0)),
            pl.BlockSpec((_RB, 1), lambda i: (i, 0)),
        ],
        out_specs=[
            pl.BlockSpec((_RB, 1), lambda i: (i, 0)),
            pl.BlockSpec((_RB, 1), lambda i: (i, 0)),
            pl.BlockSpec((_RB, 1), lambda i: (i, 0)),
            pl.BlockSpec((48, 16), lambda i: (0, 0)),
        ],
        out_shape=[
            jax.ShapeDtypeStruct((_B, 1), jnp.float32),
            jax.ShapeDtypeStruct((_B, 1), jnp.float32),
            jax.ShapeDtypeStruct((_B, 1), jnp.float32),
            jax.ShapeDtypeStruct((48, 16), jnp.float32),
        ],
    )(x, y.reshape(_B, 1))


_NSUB = 16
_CHUNK = _B // _NSUB
_NV = _CHUNK // 16


def _sc_loss_kernel(conf_hbm, acc_hbm, tl_hbm, btab_hbm, out_hbm,
                    conf_v, acc_v, tl_v, btab_v,
                    part_v, all_v, res_v):
    wid = lax.axis_index("s")
    base = wid * _CHUNK
    pltpu.sync_copy(conf_hbm.at[pl.ds(base, _CHUNK)], conf_v)
    pltpu.sync_copy(acc_hbm.at[pl.ds(base, _CHUNK)], acc_v)
    pltpu.sync_copy(tl_hbm.at[pl.ds(base, _CHUNK)], tl_v)
    pltpu.sync_copy(btab_hbm, btab_v)

    onef = jnp.ones((16,), jnp.float32)
    zerof = jnp.zeros((16,), jnp.float32)
    gam = jnp.full((16,), _GAMMA, jnp.float32)
    nrow = [btab_v[pl.ds(b * 16, 16)] for b in range(_NUM_BINS)]
    sarow = [btab_v[pl.ds((16 + b) * 16, 16)] for b in range(_NUM_BINS)]
    scrow = [btab_v[pl.ds((32 + b) * 16, 16)] for b in range(_NUM_BINS)]

    part = jnp.zeros((16,), jnp.float32)
    for i in range(_NV):
        sl = pl.ds(i * 16, 16)
        c = conf_v[sl]
        a = acc_v[sl]
        t = tl_v[sl]
        n = zerof
        sa = zerof
        sc = zerof
        for b in range(_NUM_BINS):
            m = jnp.logical_and(c > float(_LOWERS[b]), c <= float(_UPPERS[b]))
            n = jnp.where(m, nrow[b], n)
            sa = jnp.where(m, sarow[b], sa)
            sc = jnp.where(m, scrow[b], sc)
        n_safe = jnp.maximum(n, onef)
        bin_err = jnp.abs(sc / n_safe - sa / n_safe)
        n1 = n - onef
        n1_safe = jnp.maximum(n1, onef)
        acc_loo = (sa - a) / n1_safe
        conf_loo = (sc - c) / n1_safe
        loo = jnp.abs(conf_loo - acc_loo)
        upd = jnp.where(n1 > zerof, loo, zerof)
        ece = bin_err - upd
        part = part - (onef + gam * ece) * t

    part_v[...] = part
    pltpu.sync_copy(part_v, out_hbm.at[pl.ds(wid * 16, 16)])
    plsc.subcore_barrier()

    @pl.when(wid == 0)
    def _finish():
        pltpu.sync_copy(out_hbm.at[pl.ds(0, _NSUB * 16)], all_v)
        tot = jnp.zeros((16,), jnp.float32)
        for r in range(_NSUB):
            tot = tot + all_v[pl.ds(r * 16, 16)]
        scale = jnp.full((16,), 1.0 / _B, jnp.float32)
        res_v[...] = jnp.cumsum(tot * scale)
        pltpu.sync_copy(res_v, out_hbm.at[pl.ds(_NSUB * 16, 16)])


def _sc_loss(conf, acc, tl, btab):
    mesh = plsc.VectorSubcoreMesh(
        core_axis_name="c", subcore_axis_name="s", num_cores=1
    )
    f = pl.kernel(
        _sc_loss_kernel,
        mesh=mesh,
        out_type=jax.ShapeDtypeStruct(((_NSUB + 1) * 16,), jnp.float32),
        compiler_params=pltpu.CompilerParams(needs_layout_passes=False),
        scratch_types=[
            pltpu.VMEM((_CHUNK,), jnp.float32),
            pltpu.VMEM((_CHUNK,), jnp.float32),
            pltpu.VMEM((_CHUNK,), jnp.float32),
            pltpu.VMEM((48 * 16,), jnp.float32),
            pltpu.VMEM((16,), jnp.float32),
            pltpu.VMEM((_NSUB * 16,), jnp.float32),
            pltpu.VMEM((16,), jnp.float32),
        ],
    )
    out = f(conf.reshape(_B), acc.reshape(_B), tl.reshape(_B),
            btab.reshape(48 * 16))
    return out[_NSUB * 16 + 15]


def kernel(x, y):
    conf, acc, tl, btab = _row_stats(x, y)
    return _sc_loss(conf, acc, tl, btab)


# speedup vs baseline: 1.0074x; 1.0074x over previous
import jax
import jax.numpy as jnp
import numpy as np
from jax import lax
from jax.experimental import pallas as pl
from jax.experimental.pallas import tpu as pltpu
from jax.experimental.pallas import tpu_sc as plsc

_GAMMA = 0.047
_NUM_BINS = 15
_BOUNDS = np.linspace(0.0, 1.0, _NUM_BINS + 1).astype(np.float32)
_LOWERS = _BOUNDS[:-1]
_UPPERS = _BOUNDS[1:]

_B = 16384
_C = 1000
_RB = 512


def _stats_kernel(x_ref, y_ref, conf_ref, acc_ref, tl_ref, btab_ref):
    i = pl.program_id(0)
    xv = x_ref[...]
    yv = y_ref[...]
    m = jnp.max(xv, axis=1, keepdims=True)
    e = jnp.exp(xv - m)
    s = jnp.sum(e, axis=1, keepdims=True)
    conf = 1.0 / s
    col = jax.lax.broadcasted_iota(jnp.int32, xv.shape, 1)
    xy = jnp.sum(jnp.where(col == yv, xv, 0.0), axis=1, keepdims=True)
    accv = (xy == m).astype(jnp.float32)
    tl = xy - m - jnp.log(s)
    conf_ref[...] = conf
    acc_ref[...] = accv
    tl_ref[...] = tl

    cnt = jnp.zeros_like(conf, dtype=jnp.int32)
    for lo in _LOWERS:
        cnt = cnt + (conf > float(lo)).astype(jnp.int32)
    idx = cnt - 1
    lane = jax.lax.broadcasted_iota(jnp.int32, (_RB, 128), 1)
    onehot = (lane == idx).astype(jnp.float32)
    n_p = jnp.sum(onehot, axis=0, keepdims=True)
    sa_p = jnp.sum(onehot * accv, axis=0, keepdims=True)
    sc_p = jnp.sum(onehot * conf, axis=0, keepdims=True)
    groups = [
        jnp.broadcast_to(jnp.transpose(v[:, :16], (1, 0)), (16, 16))
        for v in (n_p, sa_p, sc_p)
    ]
    upd = jnp.concatenate(groups, axis=0)

    @pl.when(i == 0)
    def _init():
        btab_ref[...] = upd

    @pl.when(i != 0)
    def _accum():
        btab_ref[...] += upd


def _row_stats(x, y):
    grid = _B // _RB
    return pl.pallas_call(
        _stats_kernel,
        grid=(grid,),
        in_specs=[
            pl.BlockSpec((_RB, _C), lambda i: (i, 0)),
            pl.BlockSpec((_RB, 1), lambda i: (i, 0)),
        ],
        out_specs=[
            pl.BlockSpec((_RB, 1), lambda i: (i, 0)),
            pl.BlockSpec((_RB, 1), lambda i: (i, 0)),
            pl.BlockSpec((_RB, 1), lambda i: (i, 0)),
            pl.BlockSpec((48, 16), lambda i: (0, 0)),
        ],
        out_shape=[
            jax.ShapeDtypeStruct((_B, 1), jnp.float32),
            jax.ShapeDtypeStruct((_B, 1), jnp.float32),
            jax.ShapeDtypeStruct((_B, 1), jnp.float32),
            jax.ShapeDtypeStruct((48, 16), jnp.float32),
        ],
    )(x, y.reshape(_B, 1))


_NSUB = 16
_CHUNK = _B // _NSUB
_NV = _CHUNK // 16


def _sc_loss_kernel(conf_hbm, acc_hbm, tl_hbm, btab_hbm, out_hbm,
                    conf_v, acc_v, tl_v, btab_v,
                    part_v, all_v, res_v, sem0, sem1, sem2, sem3):
    wid = lax.axis_index("s")
    base = wid * _CHUNK
    cp0 = pltpu.async_copy(conf_hbm.at[pl.ds(base, _CHUNK)], conf_v, sem0)
    cp1 = pltpu.async_copy(acc_hbm.at[pl.ds(base, _CHUNK)], acc_v, sem1)
    cp2 = pltpu.async_copy(tl_hbm.at[pl.ds(base, _CHUNK)], tl_v, sem2)
    cp3 = pltpu.async_copy(btab_hbm, btab_v, sem3)
    cp0.wait()
    cp1.wait()
    cp2.wait()
    cp3.wait()

    onef = jnp.ones((16,), jnp.float32)
    zerof = jnp.zeros((16,), jnp.float32)
    gam = jnp.full((16,), _GAMMA, jnp.float32)
    nrow = [btab_v[pl.ds(b * 16, 16)] for b in range(_NUM_BINS)]
    sarow = [btab_v[pl.ds((16 + b) * 16, 16)] for b in range(_NUM_BINS)]
    scrow = [btab_v[pl.ds((32 + b) * 16, 16)] for b in range(_NUM_BINS)]

    part = jnp.zeros((16,), jnp.float32)
    for i in range(_NV):
        sl = pl.ds(i * 16, 16)
        c = conf_v[sl]
        a = acc_v[sl]
        t = tl_v[sl]
        n = zerof
        sa = zerof
        sc = zerof
        for b in range(_NUM_BINS):
            m = jnp.logical_and(c > float(_LOWERS[b]), c <= float(_UPPERS[b]))
            n = jnp.where(m, nrow[b], n)
            sa = jnp.where(m, sarow[b], sa)
            sc = jnp.where(m, scrow[b], sc)
        n_safe = jnp.maximum(n, onef)
        bin_err = jnp.abs(sc / n_safe - sa / n_safe)
        n1 = n - onef
        n1_safe = jnp.maximum(n1, onef)
        acc_loo = (sa - a) / n1_safe
        conf_loo = (sc - c) / n1_safe
        loo = jnp.abs(conf_loo - acc_loo)
        upd = jnp.where(n1 > zerof, loo, zerof)
        ece = bin_err - upd
        part = part - (onef + gam * ece) * t

    part_v[...] = part
    pltpu.sync_copy(part_v, out_hbm.at[pl.ds(wid * 16, 16)])
    plsc.subcore_barrier()

    @pl.when(wid == 0)
    def _finish():
        pltpu.sync_copy(out_hbm.at[pl.ds(0, _NSUB * 16)], all_v)
        tot = jnp.zeros((16,), jnp.float32)
        for r in range(_NSUB):
            tot = tot + all_v[pl.ds(r * 16, 16)]
        scale = jnp.full((16,), 1.0 / _B, jnp.float32)
        res_v[...] = jnp.cumsum(tot * scale)
        pltpu.sync_copy(res_v, out_hbm.at[pl.ds(_NSUB * 16, 16)])


def _sc_loss(conf, acc, tl, btab):
    mesh = plsc.VectorSubcoreMesh(
        core_axis_name="c", subcore_axis_name="s", num_cores=1
    )
    f = pl.kernel(
        _sc_loss_kernel,
        mesh=mesh,
        out_type=jax.ShapeDtypeStruct(((_NSUB + 1) * 16,), jnp.float32),
        compiler_params=pltpu.CompilerParams(needs_layout_passes=False),
        scratch_types=[
            pltpu.VMEM((_CHUNK,), jnp.float32),
            pltpu.VMEM((_CHUNK,), jnp.float32),
            pltpu.VMEM((_CHUNK,), jnp.float32),
            pltpu.VMEM((48 * 16,), jnp.float32),
            pltpu.VMEM((16,), jnp.float32),
            pltpu.VMEM((_NSUB * 16,), jnp.float32),
            pltpu.VMEM((16,), jnp.float32),
            pltpu.SemaphoreType.DMA,
            pltpu.SemaphoreType.DMA,
            pltpu.SemaphoreType.DMA,
            pltpu.SemaphoreType.DMA,
        ],
    )
    out = f(conf.reshape(_B), acc.reshape(_B), tl.reshape(_B),
            btab.reshape(48 * 16))
    return out[_NSUB * 16 + 15]


def kernel(x, y):
    conf, acc, tl, btab = _row_stats(x, y)
    return _sc_loss(conf, acc, tl, btab)
